# Initial kernel scaffold; baseline (speedup 1.0000x reference)
import functools

import jax
import jax.numpy as jnp
from jax import lax
from jax.experimental import pallas as pl
from jax.experimental.pallas import tpu as pltpu
from jax.experimental.pallas import tpu_sc as plsc

N = 10000
E = 320000
D = 128

NC = 2
NS = 16
NW = NC * NS
CH = 128
JCH = 79
EPW = JCH * CH
E_PAD = NW * EPW

NJ_FULL = 78
TAIL_BASE = NJ_FULL * CH
TAIL = N - TAIL_BASE

_mesh = plsc.VectorSubcoreMesh(
    core_axis_name="c", subcore_axis_name="s", num_cores=NC, num_subcores=NS)


def _worker(c, s):
    return c * NS + s


@functools.partial(
    pl.kernel,
    out_type=jax.ShapeDtypeStruct((NC, N), jnp.float32),
    mesh=_mesh,
    scratch_types=[
        pltpu.VMEM((JCH, CH), jnp.int32),
        pltpu.VMEM((JCH, CH), jnp.float32),
        pltpu.VMEM((CH,), jnp.float32),
        pltpu.VMEM_SHARED((N,), jnp.float32),
    ],
)
def _deg_kernel(dst_hbm, ew_hbm, out_hbm, dst_v, ew_v, zb_v, deg_sh):
    c = lax.axis_index("c")
    s = lax.axis_index("s")
    w = _worker(c, s)
    zeros16 = jnp.zeros((16,), jnp.float32)
    for g in range(CH // 16):
        zb_v[pl.ds(g * 16, 16)] = zeros16
    for jj in range(5):
        j = s + NS * jj

        @pl.when(j < NJ_FULL)
        def _():
            pltpu.sync_copy(zb_v, deg_sh.at[pl.ds(j * CH, CH)])

        @pl.when(j == NJ_FULL)
        def _():
            pltpu.sync_copy(zb_v.at[pl.ds(0, TAIL)],
                            deg_sh.at[pl.ds(TAIL_BASE, TAIL)])

    plsc.subcore_barrier()
    pltpu.sync_copy(dst_hbm.at[w], dst_v)
    pltpu.sync_copy(ew_hbm.at[w], ew_v)

    def body(j, carry):
        pltpu.sync_copy(ew_v.at[j], deg_sh.at[dst_v.at[j]], add=True)
        return carry

    lax.fori_loop(0, JCH, body, 0)
    plsc.subcore_barrier()
    for jj in range(5):
        j = s + NS * jj

        @pl.when(j < NJ_FULL)
        def _():
            pltpu.sync_copy(deg_sh.at[pl.ds(j * CH, CH)],
                            out_hbm.at[c, pl.ds(j * CH, CH)])

        @pl.when(j == NJ_FULL)
        def _():
            pltpu.sync_copy(deg_sh.at[pl.ds(TAIL_BASE, TAIL)],
                            out_hbm.at[c, pl.ds(TAIL_BASE, TAIL)])


@functools.partial(
    pl.kernel,
    out_type=jax.ShapeDtypeStruct((NC, N, D), jnp.float32),
    mesh=_mesh,
    scratch_types=[
        pltpu.VMEM((JCH, CH), jnp.int32),
        pltpu.VMEM((JCH, CH), jnp.int32),
        pltpu.VMEM((CH, D), jnp.float32),
        pltpu.SMEM((CH,), jnp.float32),
        pltpu.SemaphoreType.DMA,
        pltpu.VMEM_SHARED((N, D), jnp.float32),
    ],
)
def _agg_kernel(hs_hbm, src_hbm, dst_hbm, ew_hbm, out_hbm,
                src_v, dst_v, rows_v, ew_s, gsem, acc_sh):
    c = lax.axis_index("c")
    s = lax.axis_index("s")
    w = _worker(c, s)
    zeros16 = jnp.zeros((16,), jnp.float32)
    for r in range(CH):
        for g in range(D // 16):
            rows_v[r, pl.ds(g * 16, 16)] = zeros16
    for jj in range(5):
        j = s + NS * jj

        @pl.when(j < NJ_FULL)
        def _():
            pltpu.sync_copy(rows_v, acc_sh.at[pl.ds(j * CH, CH)])

        @pl.when(j == NJ_FULL)
        def _():
            pltpu.sync_copy(rows_v.at[pl.ds(0, TAIL)],
                            acc_sh.at[pl.ds(TAIL_BASE, TAIL)])

    plsc.subcore_barrier()
    pltpu.sync_copy(src_hbm.at[w], src_v)
    pltpu.sync_copy(dst_hbm.at[w], dst_v)

    def body(j, carry):
        pltpu.sync_copy(ew_hbm.at[w, j], ew_s)
        pltpu.async_copy(hs_hbm.at[src_v.at[j]], rows_v, gsem).wait()
        for e in range(CH):
            wv = jnp.full((16,), ew_s[e], dtype=jnp.float32)
            for g in range(D // 16):
                sl = pl.ds(g * 16, 16)
                rows_v[e, sl] = rows_v[e, sl] * wv
        pltpu.sync_copy(rows_v, acc_sh.at[dst_v.at[j]], add=True)
        return carry

    lax.fori_loop(0, JCH, body, 0)
    plsc.subcore_barrier()
    for jj in range(5):
        j = s + NS * jj

        @pl.when(j < NJ_FULL)
        def _():
            pltpu.sync_copy(acc_sh.at[pl.ds(j * CH, CH)],
                            out_hbm.at[c, pl.ds(j * CH, CH)])

        @pl.when(j == NJ_FULL)
        def _():
            pltpu.sync_copy(acc_sh.at[pl.ds(TAIL_BASE, TAIL)],
                            out_hbm.at[c, pl.ds(TAIL_BASE, TAIL)])


ROWS_BLK = 1000
GRID = N // ROWS_BLK


def _dis_block(degp):
    deg = degp[0, :] + degp[1, :] + 1.0
    return lax.rsqrt(deg)


def _t_first_body(degp_ref, x_ref, w_ref, o_ref):
    dis = _dis_block(degp_ref[...])
    h = jnp.dot(x_ref[...], w_ref[...], preferred_element_type=jnp.float32)
    o_ref[...] = h * dis[:, None]


def _t_mid_body(degp_ref, p_ref, hprev_ref, b_ref, w_ref, o_ref):
    dis = _dis_block(degp_ref[...])
    agg = (p_ref[0] + p_ref[1] + hprev_ref[...]) * dis[:, None]
    y = jnp.maximum(agg + b_ref[...][None, :], 0.0)
    h = jnp.dot(y, w_ref[...], preferred_element_type=jnp.float32)
    o_ref[...] = h * dis[:, None]


def _t_final_body(degp_ref, p_ref, hprev_ref, b_ref, o_ref):
    dis = _dis_block(degp_ref[...])
    agg = (p_ref[0] + p_ref[1] + hprev_ref[...]) * dis[:, None]
    o_ref[...] = jnp.maximum(agg + b_ref[...][None, :], 0.0)


_degp_spec = pl.BlockSpec((NC, ROWS_BLK), lambda i: (0, i))
_p_spec = pl.BlockSpec((NC, ROWS_BLK, D), lambda i: (0, i, 0))
_rows_spec = pl.BlockSpec((ROWS_BLK, D), lambda i: (i, 0))
_w_spec = pl.BlockSpec((D, D), lambda i: (0, 0))
_b_spec = pl.BlockSpec((D,), lambda i: (0,))
_out_shape = jax.ShapeDtypeStruct((N, D), jnp.float32)

_t_first = pl.pallas_call(
    _t_first_body, grid=(GRID,),
    in_specs=[_degp_spec, _rows_spec, _w_spec],
    out_specs=_rows_spec, out_shape=_out_shape)

_t_mid = pl.pallas_call(
    _t_mid_body, grid=(GRID,),
    in_specs=[_degp_spec, _p_spec, _rows_spec, _b_spec, _w_spec],
    out_specs=_rows_spec, out_shape=_out_shape)

_t_final = pl.pallas_call(
    _t_final_body, grid=(GRID,),
    in_specs=[_degp_spec, _p_spec, _rows_spec, _b_spec],
    out_specs=_rows_spec, out_shape=_out_shape)


def kernel(x, edge_index, edge_weight, W1, b1, W2, b2, W3, b3):
    src = edge_index[0]
    dst = edge_index[1]
    pad = E_PAD - E
    pad_idx = (jnp.arange(pad, dtype=jnp.int32) * 37) % N
    src_p = jnp.concatenate([src, pad_idx]).reshape(NW, JCH, CH)
    dst_p = jnp.concatenate([dst, pad_idx]).reshape(NW, JCH, CH)
    ew_p = jnp.concatenate(
        [edge_weight, jnp.zeros((pad,), jnp.float32)]).reshape(NW, JCH, CH)

    degp = _deg_kernel(dst_p, ew_p)
    h1s = _t_first(degp, x, W1)
    p1 = _agg_kernel(h1s, src_p, dst_p, ew_p)
    h2s = _t_mid(degp, p1, h1s, b1, W2)
    p2 = _agg_kernel(h2s, src_p, dst_p, ew_p)
    h3s = _t_mid(degp, p2, h2s, b2, W3)
    p3 = _agg_kernel(h3s, src_p, dst_p, ew_p)
    return _t_final(degp, p3, h3s, b3)


# speedup vs baseline: 14.4011x; 14.4011x over previous
import functools

import jax
import jax.numpy as jnp
from jax import lax
from jax.experimental import pallas as pl
from jax.experimental.pallas import tpu as pltpu
from jax.experimental.pallas import tpu_sc as plsc

N = 10000
E = 320000
D = 128

NC = 2
NS = 16
NW = NC * NS
CH = 128
JCH = 79
EPW = JCH * CH
E_PAD = NW * EPW

NJ = 79
N_PAD = NJ * CH

_mesh = plsc.VectorSubcoreMesh(
    core_axis_name="c", subcore_axis_name="s", num_cores=NC, num_subcores=NS)


def _worker(c, s):
    return c * NS + s


@functools.partial(
    pl.kernel,
    out_type=jax.ShapeDtypeStruct((NC, N_PAD), jnp.float32),
    mesh=_mesh,
    scratch_types=[
        pltpu.VMEM((JCH, CH), jnp.int32),
        pltpu.VMEM((JCH, CH), jnp.float32),
        pltpu.VMEM((CH,), jnp.float32),
        pltpu.VMEM_SHARED((N_PAD,), jnp.float32),
    ],
)
def _deg_kernel(dst_hbm, ew_hbm, out_hbm, dst_v, ew_v, zb_v, deg_sh):
    c = lax.axis_index("c")
    s = lax.axis_index("s")
    w = _worker(c, s)
    zeros16 = jnp.zeros((16,), jnp.float32)
    for g in range(CH // 16):
        zb_v[pl.ds(g * 16, 16)] = zeros16
    for jj in range(5):
        j = s + NS * jj

        @pl.when(j < NJ)
        def _():
            pltpu.sync_copy(zb_v, deg_sh.at[pl.ds(j * CH, CH)])

    plsc.subcore_barrier()
    pltpu.sync_copy(dst_hbm.at[w], dst_v)
    pltpu.sync_copy(ew_hbm.at[w], ew_v)

    def body(j, carry):
        pltpu.sync_copy(ew_v.at[j], deg_sh.at[dst_v.at[j]], add=True)
        return carry

    lax.fori_loop(0, JCH, body, 0)
    plsc.subcore_barrier()
    for jj in range(5):
        j = s + NS * jj

        @pl.when(j < NJ)
        def _():
            pltpu.sync_copy(deg_sh.at[pl.ds(j * CH, CH)],
                            out_hbm.at[c, pl.ds(j * CH, CH)])


@functools.partial(
    pl.kernel,
    out_type=jax.ShapeDtypeStruct((NC, N_PAD, D), jnp.float32),
    mesh=_mesh,
    scratch_types=[
        pltpu.VMEM((JCH, CH), jnp.int32),
        pltpu.VMEM((JCH, CH), jnp.int32),
        pltpu.VMEM((CH, D), jnp.float32),
        pltpu.SMEM((CH,), jnp.float32),
        pltpu.SemaphoreType.DMA,
        pltpu.VMEM_SHARED((NS * EPW,), jnp.float32),
        pltpu.VMEM_SHARED((N_PAD, D), jnp.float32),
    ],
)
def _agg_kernel(hs_hbm, src_hbm, dst_hbm, ew_hbm, out_hbm,
                src_v, dst_v, rows_v, ew_s, gsem, ew_sh, acc_sh):
    c = lax.axis_index("c")
    s = lax.axis_index("s")
    w = _worker(c, s)
    zeros16 = jnp.zeros((16,), jnp.float32)
    for r in range(CH):
        for g in range(D // 16):
            rows_v[r, pl.ds(g * 16, 16)] = zeros16
    for jj in range(5):
        j = s + NS * jj

        @pl.when(j < NJ)
        def _():
            pltpu.sync_copy(rows_v, acc_sh.at[pl.ds(j * CH, CH)])

    plsc.subcore_barrier()
    pltpu.sync_copy(src_hbm.at[w], src_v)
    pltpu.sync_copy(dst_hbm.at[w], dst_v)
    pltpu.sync_copy(ew_hbm.at[w], ew_sh.at[pl.ds(s * EPW, EPW)])

    def body(j, carry):
        pltpu.sync_copy(ew_sh.at[pl.ds(s * EPW + j * CH, CH)], ew_s)
        pltpu.async_copy(hs_hbm.at[src_v.at[j]], rows_v, gsem).wait()
        for e in range(CH):
            wv = jnp.full((16,), ew_s[e], dtype=jnp.float32)
            for g in range(D // 16):
                sl = pl.ds(g * 16, 16)
                rows_v[e, sl] = rows_v[e, sl] * wv
        pltpu.sync_copy(rows_v, acc_sh.at[dst_v.at[j]], add=True)
        return carry

    lax.fori_loop(0, JCH, body, 0)
    plsc.subcore_barrier()
    for jj in range(5):
        j = s + NS * jj

        @pl.when(j < NJ)
        def _():
            pltpu.sync_copy(acc_sh.at[pl.ds(j * CH, CH)],
                            out_hbm.at[c, pl.ds(j * CH, CH)])


ROWS_BLK = 1000
GRID = N // ROWS_BLK


def _dis_block(degp):
    deg = degp[:, 0] + degp[:, 1] + 1.0
    return lax.rsqrt(deg)


def _t_first_body(degp_ref, x_ref, w_ref, o_ref):
    dis = _dis_block(degp_ref[...])
    h = jnp.dot(x_ref[...], w_ref[...], preferred_element_type=jnp.float32)
    o_ref[...] = h * dis[:, None]


def _t_mid_body(degp_ref, p_ref, hprev_ref, b_ref, w_ref, o_ref):
    dis = _dis_block(degp_ref[...])
    agg = (p_ref[0] + p_ref[1] + hprev_ref[...]) * dis[:, None]
    y = jnp.maximum(agg + b_ref[...][None, :], 0.0)
    h = jnp.dot(y, w_ref[...], preferred_element_type=jnp.float32)
    o_ref[...] = h * dis[:, None]


def _t_final_body(degp_ref, p_ref, hprev_ref, b_ref, o_ref):
    dis = _dis_block(degp_ref[...])
    agg = (p_ref[0] + p_ref[1] + hprev_ref[...]) * dis[:, None]
    o_ref[...] = jnp.maximum(agg + b_ref[...][None, :], 0.0)


_degp_spec = pl.BlockSpec((ROWS_BLK, NC), lambda i: (i, 0))
_p_spec = pl.BlockSpec((NC, ROWS_BLK, D), lambda i: (0, i, 0))
_rows_spec = pl.BlockSpec((ROWS_BLK, D), lambda i: (i, 0))
_w_spec = pl.BlockSpec((D, D), lambda i: (0, 0))
_b_spec = pl.BlockSpec((D,), lambda i: (0,))
_out_shape = jax.ShapeDtypeStruct((N, D), jnp.float32)

_t_first = pl.pallas_call(
    _t_first_body, grid=(GRID,),
    in_specs=[_degp_spec, _rows_spec, _w_spec],
    out_specs=_rows_spec, out_shape=_out_shape)

_t_mid = pl.pallas_call(
    _t_mid_body, grid=(GRID,),
    in_specs=[_degp_spec, _p_spec, _rows_spec, _b_spec, _w_spec],
    out_specs=_rows_spec, out_shape=_out_shape)

_t_final = pl.pallas_call(
    _t_final_body, grid=(GRID,),
    in_specs=[_degp_spec, _p_spec, _rows_spec, _b_spec],
    out_specs=_rows_spec, out_shape=_out_shape)


def kernel(x, edge_index, edge_weight, W1, b1, W2, b2, W3, b3):
    src = edge_index[0]
    dst = edge_index[1]
    pad = E_PAD - E
    pad_idx = (jnp.arange(pad, dtype=jnp.int32) * 37) % N
    src_p = jnp.concatenate([src, pad_idx]).reshape(NW, JCH, CH)
    dst_p = jnp.concatenate([dst, pad_idx]).reshape(NW, JCH, CH)
    ew_p = jnp.concatenate(
        [edge_weight, jnp.zeros((pad,), jnp.float32)]).reshape(NW, JCH, CH)

    ew_flat = ew_p.reshape(NW, EPW)
    degp = _deg_kernel(dst_p, ew_p)[:, :N].T
    h1s = _t_first(degp, x, W1)
    p1 = _agg_kernel(h1s, src_p, dst_p, ew_flat)[:, :N]
    h2s = _t_mid(degp, p1, h1s, b1, W2)
    p2 = _agg_kernel(h2s, src_p, dst_p, ew_flat)[:, :N]
    h3s = _t_mid(degp, p2, h2s, b2, W3)
    p3 = _agg_kernel(h3s, src_p, dst_p, ew_flat)[:, :N]
    return _t_final(degp, p3, h3s, b3)


# speedup vs baseline: 24.1076x; 1.6740x over previous
import functools

import jax
import jax.numpy as jnp
from jax import lax
from jax.experimental import pallas as pl
from jax.experimental.pallas import tpu as pltpu
from jax.experimental.pallas import tpu_sc as plsc

N = 10000
E = 320000
D = 128

NC = 2
NS = 16
NW = NC * NS
CH = 128
NBUF = 3
IR = 4

JCH = 81
EPW = JCH * CH
E_PAD = NW * EPW

NJF = 78
TAILB = NJF * CH
TAIL = N - TAILB

DJCH = 79
DEPW = DJCH * CH
DE_PAD = NW * DEPW
DCH = 128
DNJ = 79
DEG_PAD = DNJ * DCH

_mesh = plsc.VectorSubcoreMesh(
    core_axis_name="c", subcore_axis_name="s", num_cores=NC, num_subcores=NS)


def _worker(c, s):
    return c * NS + s


_DEG_KW = dict(
    out_type=jax.ShapeDtypeStruct((NC, DEG_PAD), jnp.float32),
    mesh=_mesh,
    scratch_types=[
        pltpu.VMEM((DJCH, CH), jnp.int32),
        pltpu.VMEM((DJCH, CH), jnp.float32),
        pltpu.VMEM((DCH,), jnp.float32),
        pltpu.VMEM_SHARED((DEG_PAD,), jnp.float32),
    ],
)


def _deg_body(dst_hbm, ew_hbm, out_hbm, dst_v, ew_v, zb_v, deg_sh):
    c = lax.axis_index("c")
    s = lax.axis_index("s")
    w = _worker(c, s)
    zeros16 = jnp.zeros((16,), jnp.float32)
    for g in range(DCH // 16):
        zb_v[pl.ds(g * 16, 16)] = zeros16
    for jj in range(5):
        j = s + NS * jj

        @pl.when(j < DNJ)
        def _():
            pltpu.sync_copy(zb_v, deg_sh.at[pl.ds(j * DCH, DCH)])

    plsc.subcore_barrier()
    pltpu.sync_copy(dst_hbm.at[w], dst_v)
    pltpu.sync_copy(ew_hbm.at[w], ew_v)

    def body(j, carry):
        pltpu.sync_copy(ew_v.at[j], deg_sh.at[dst_v.at[j]], add=True)
        return carry

    lax.fori_loop(0, DJCH, body, 0)
    plsc.subcore_barrier()
    for jj in range(5):
        j = s + NS * jj

        @pl.when(j < DNJ)
        def _():
            pltpu.sync_copy(deg_sh.at[pl.ds(j * DCH, DCH)],
                            out_hbm.at[c, pl.ds(j * DCH, DCH)])


_deg_kernel = pl.kernel(_deg_body, **_DEG_KW)


_AGG_KW = dict(
    out_type=jax.ShapeDtypeStruct((NC, N, D), jnp.float32),
    mesh=_mesh,
    scratch_types=[
        pltpu.VMEM((NBUF, CH, D), jnp.float32),
        pltpu.VMEM((IR, CH), jnp.int32),
        pltpu.VMEM((IR, CH), jnp.int32),
        pltpu.SMEM((CH,), jnp.float32),
        pltpu.SemaphoreType.DMA((NBUF,)),
        pltpu.SemaphoreType.DMA((NBUF,)),
        pltpu.SemaphoreType.DMA((IR,)),
        pltpu.SemaphoreType.DMA((IR,)),
        pltpu.SemaphoreType.DMA((IR,)),
        pltpu.VMEM_SHARED((NS * IR, CH), jnp.float32),
        pltpu.VMEM_SHARED((N, D), jnp.float32),
    ],
)


def _agg_body(hs_hbm, src_hbm, dst_hbm, ew_hbm, out_hbm,
              rows_v, src_v, dst_v, ew_s, gsem, ssem, isem, dsem, esem,
              ew_sh, acc_sh):
    c = lax.axis_index("c")
    s = lax.axis_index("s")
    w = _worker(c, s)
    zeros16 = jnp.zeros((16,), jnp.float32)
    zb = rows_v.at[0]
    for r in range(CH):
        for g in range(D // 16):
            zb[r, pl.ds(g * 16, 16)] = zeros16
    for jj in range(5):
        j = s + NS * jj

        @pl.when(j < NJF)
        def _():
            pltpu.sync_copy(zb, acc_sh.at[pl.ds(j * CH, CH)])

        @pl.when(j == NJF)
        def _():
            pltpu.sync_copy(zb.at[pl.ds(0, TAIL)],
                            acc_sh.at[pl.ds(TAILB, TAIL)])

    plsc.subcore_barrier()

    def start_ring(j):
        k = j % IR
        pltpu.async_copy(src_hbm.at[w * JCH + j], src_v.at[k], isem.at[k])
        pltpu.async_copy(dst_hbm.at[w * JCH + j], dst_v.at[k], dsem.at[k])
        pltpu.async_copy(ew_hbm.at[w * JCH + j], ew_sh.at[s * IR + k],
                         esem.at[k])

    def wait_src(j):
        k = j % IR
        pltpu.make_async_copy(
            src_hbm.at[w * JCH + j], src_v.at[k], isem.at[k]).wait()

    def start_gather(j, b):
        pltpu.async_copy(hs_hbm.at[src_v.at[j % IR]], rows_v.at[b],
                         gsem.at[b])

    def _scale(j, b):
        k = j % IR
        pltpu.make_async_copy(
            ew_hbm.at[w * JCH + j], ew_sh.at[s * IR + k], esem.at[k]).wait()
        pltpu.sync_copy(ew_sh.at[s * IR + k], ew_s)
        buf = rows_v.at[b]

        def escale(e, carry):
            for u in range(8):
                wv = jnp.full((16,), ew_s[e * 8 + u], dtype=jnp.float32)
                for g in range(D // 16):
                    sl = pl.ds(g * 16, 16)
                    buf[e * 8 + u, sl] = buf[e * 8 + u, sl] * wv
            return carry

        lax.fori_loop(0, CH // 8, escale, 0)

    for k in range(NBUF):
        start_ring(k)
    for b in range(NBUF - 1):
        wait_src(b)
        start_gather(b, b)

    def outer(g_it, carry):
        for b in range(NBUF):
            j = NBUF * g_it + b
            bp = (b + NBUF - 1) % NBUF
            pltpu.make_async_copy(
                hs_hbm.at[src_v.at[j % IR]], rows_v.at[b], gsem.at[b]).wait()

            @pl.when(j >= 1)
            def _():
                pltpu.make_async_copy(
                    rows_v.at[bp], acc_sh.at[dst_v.at[(j - 1) % IR]],
                    ssem.at[bp]).wait()

            @pl.when(j + NBUF < JCH)
            def _():
                start_ring(j + NBUF)

            @pl.when(j + NBUF - 1 < JCH)
            def _():
                wait_src(j + NBUF - 1)
                start_gather(j + NBUF - 1, bp)

            _scale(j, b)
            pltpu.make_async_copy(
                dst_hbm.at[w * JCH + j], dst_v.at[j % IR],
                dsem.at[j % IR]).wait()
            pltpu.async_copy(
                rows_v.at[b], acc_sh.at[dst_v.at[j % IR]], ssem.at[b],
                add=True)

        return carry

    lax.fori_loop(0, JCH // NBUF, outer, 0)
    pltpu.make_async_copy(
        rows_v.at[(JCH - 1) % NBUF], acc_sh.at[dst_v.at[(JCH - 1) % IR]],
        ssem.at[(JCH - 1) % NBUF]).wait()
    plsc.subcore_barrier()
    for jj in range(5):
        j = s + NS * jj

        @pl.when(j < NJF)
        def _():
            pltpu.sync_copy(acc_sh.at[pl.ds(j * CH, CH)],
                            out_hbm.at[c, pl.ds(j * CH, CH)])

        @pl.when(j == NJF)
        def _():
            pltpu.sync_copy(acc_sh.at[pl.ds(TAILB, TAIL)],
                            out_hbm.at[c, pl.ds(TAILB, TAIL)])


_agg_kernel = pl.kernel(_agg_body, **_AGG_KW)


ROWS_BLK = 1000
GRID = N // ROWS_BLK


def _dis_block(degp):
    deg = degp[:, 0] + degp[:, 1] + 1.0
    return lax.rsqrt(deg)


def _psum(p_ref):
    return p_ref[0] + p_ref[1]


def _t_first_body(degp_ref, x_ref, w_ref, o_ref):
    dis = _dis_block(degp_ref[...])
    h = jnp.dot(x_ref[...], w_ref[...], preferred_element_type=jnp.float32)
    o_ref[...] = h * dis[:, None]


def _t_mid_body(degp_ref, p_ref, hprev_ref, b_ref, w_ref, o_ref):
    dis = _dis_block(degp_ref[...])
    agg = (_psum(p_ref) + hprev_ref[...]) * dis[:, None]
    y = jnp.maximum(agg + b_ref[...][None, :], 0.0)
    h = jnp.dot(y, w_ref[...], preferred_element_type=jnp.float32)
    o_ref[...] = h * dis[:, None]


def _t_final_body(degp_ref, p_ref, hprev_ref, b_ref, o_ref):
    dis = _dis_block(degp_ref[...])
    agg = (_psum(p_ref) + hprev_ref[...]) * dis[:, None]
    o_ref[...] = jnp.maximum(agg + b_ref[...][None, :], 0.0)


_degp_spec = pl.BlockSpec((ROWS_BLK, NC), lambda i: (i, 0))
_p_spec = pl.BlockSpec((NC, ROWS_BLK, D), lambda i: (0, i, 0))
_rows_spec = pl.BlockSpec((ROWS_BLK, D), lambda i: (i, 0))
_w_spec = pl.BlockSpec((D, D), lambda i: (0, 0))
_b_spec = pl.BlockSpec((D,), lambda i: (0,))
_out_shape = jax.ShapeDtypeStruct((N, D), jnp.float32)

_t_first = pl.pallas_call(
    _t_first_body, grid=(GRID,),
    in_specs=[_degp_spec, _rows_spec, _w_spec],
    out_specs=_rows_spec, out_shape=_out_shape)

_t_mid = pl.pallas_call(
    _t_mid_body, grid=(GRID,),
    in_specs=[_degp_spec, _p_spec, _rows_spec, _b_spec, _w_spec],
    out_specs=_rows_spec, out_shape=_out_shape)

_t_final = pl.pallas_call(
    _t_final_body, grid=(GRID,),
    in_specs=[_degp_spec, _p_spec, _rows_spec, _b_spec],
    out_specs=_rows_spec, out_shape=_out_shape)


def kernel(x, edge_index, edge_weight, W1, b1, W2, b2, W3, b3):
    src = edge_index[0]
    dst = edge_index[1]
    pad = E_PAD - E
    pad_idx = (jnp.arange(pad, dtype=jnp.int32) * 37) % N
    src_f = jnp.concatenate([src, pad_idx]).reshape(NW * JCH, CH)
    dst_f = jnp.concatenate([dst, pad_idx]).reshape(NW * JCH, CH)
    ew_f = jnp.concatenate(
        [edge_weight, jnp.zeros((pad,), jnp.float32)]).reshape(NW * JCH, CH)

    dpad = DE_PAD - E
    dpad_idx = (jnp.arange(dpad, dtype=jnp.int32) * 37) % N
    dst_d = jnp.concatenate([dst, dpad_idx]).reshape(NW, DJCH, CH)
    ew_d = jnp.concatenate(
        [edge_weight, jnp.zeros((dpad,), jnp.float32)]).reshape(NW, DJCH, CH)

    degp = _deg_kernel(dst_d, ew_d)[:, :N].T
    h1s = _t_first(degp, x, W1)
    p1 = _agg_kernel(h1s, src_f, dst_f, ew_f)
    h2s = _t_mid(degp, p1, h1s, b1, W2)
    p2 = _agg_kernel(h2s, src_f, dst_f, ew_f)
    h3s = _t_mid(degp, p2, h2s, b2, W3)
    p3 = _agg_kernel(h3s, src_f, dst_f, ew_f)
    return _t_final(degp, p3, h3s, b3)
